# baseline (speedup 1.0000x reference)
import jax
import jax.numpy as jnp
from jax import lax
from jax.experimental import pallas as pl
from jax.experimental.pallas import tpu as pltpu

D_MODEL = 768
NUM_CLASSES = 10
K = 16
TILE_T = 4096


def _body(h_ref, wc_ref, bc_ref, ws_ref, logits_ref, scores_ref, pooled_ref,
          sc_s1, sc_s8, sc_logits, sc_pool):
    b = pl.program_id(0)
    t = pl.program_id(1)
    B = pl.num_programs(0)
    nt = pl.num_programs(1)
    T = nt * TILE_T

    h = h_ref[0]
    res = jnp.dot(h, wc_ref[...], preferred_element_type=jnp.float32)
    res = res + bc_ref[...]
    logits_ref[0] = res[:, :NUM_CLASSES]
    srow = jax.lax.dot_general(
        ws_ref[...], h, (((1,), (1,)), ((), ())),
        preferred_element_type=jnp.float32) + bc_ref[0, NUM_CLASSES]
    scores_ref[0] = srow
    sc_s1[:, pl.ds(t * TILE_T, TILE_T)] = srow
    sc_logits[pl.ds(t * TILE_T, TILE_T), :] = res

    @pl.when(t == nt - 1)
    def _pool():
        W = T // 8
        neg = jnp.float32(-jnp.inf)
        sc_s8[...] = jnp.reshape(sc_s1[...], (8, W))
        iota2 = (lax.broadcasted_iota(jnp.int32, (8, W), 1)
                 + W * lax.broadcasted_iota(jnp.int32, (8, W), 0))

        def step(_, acc):
            s = sc_s8[...]
            mx = jnp.max(s)
            i = jnp.min(jnp.where(s == mx, iota2, T))
            sc_s8[...] = jnp.where(iota2 == i, neg, s)
            return acc + sc_logits[pl.ds(i, 1), :]

        acc = lax.fori_loop(0, K, step, jnp.zeros((1, 16), jnp.float32))
        row8 = pl.multiple_of(b * 8, 8)
        sc_pool[pl.ds(row8, 8), :] = jnp.broadcast_to(acc * (1.0 / K), (8, 16))

        @pl.when(b == B - 1)
        def _emit():
            rows = [sc_pool[bb * 8:bb * 8 + 1, :] for bb in range(B)]
            pooled_ref[0] = jnp.concatenate(rows, 0)


def _fused(H, W_cls, b_cls, W_score, b_score):
    B, T, D = H.shape
    nt = T // TILE_T
    wc = jnp.zeros((D, 16), jnp.float32)
    wc = wc.at[:, :NUM_CLASSES].set(W_cls.T)
    wc = wc.at[:, NUM_CLASSES:NUM_CLASSES + 1].set(W_score.T)
    bc = jnp.zeros((1, 16), jnp.float32)
    bc = bc.at[0, :NUM_CLASSES].set(b_cls).at[0, NUM_CLASSES].set(b_score[0])
    return pl.pallas_call(
        _body,
        grid=(B, nt),
        in_specs=[
            pl.BlockSpec((1, TILE_T, D), lambda b, t: (b, t, 0)),
            pl.BlockSpec((D, 16), lambda b, t: (0, 0)),
            pl.BlockSpec((1, 16), lambda b, t: (0, 0)),
            pl.BlockSpec((1, D), lambda b, t: (0, 0)),
        ],
        out_specs=[
            pl.BlockSpec((1, TILE_T, NUM_CLASSES), lambda b, t: (b, t, 0)),
            pl.BlockSpec((1, 1, TILE_T), lambda b, t: (b, 0, t)),
            pl.BlockSpec((1, B, 16), lambda b, t: (0, 0, 0)),
        ],
        out_shape=[
            jax.ShapeDtypeStruct((B, T, NUM_CLASSES), jnp.float32),
            jax.ShapeDtypeStruct((B, 1, T), jnp.float32),
            jax.ShapeDtypeStruct((1, B, 16), jnp.float32),
        ],
        scratch_shapes=[
            pltpu.VMEM((1, T), jnp.float32),
            pltpu.VMEM((8, T // 8), jnp.float32),
            pltpu.VMEM((T, 16), jnp.float32),
            pltpu.VMEM((8 * B, 16), jnp.float32),
        ],
    )(H, wc, bc, W_score)


def kernel(H, W_cls, b_cls, W_score, b_score):
    B, T, _ = H.shape
    logits_t, scores3, pooled16 = _fused(H, W_cls, b_cls, W_score, b_score)
    return (pooled16[0, :, :NUM_CLASSES], logits_t, scores3.reshape(B, T))


# speedup vs baseline: 1.1903x; 1.1903x over previous
import jax
import jax.numpy as jnp
from jax import lax
from jax.experimental import pallas as pl
from jax.experimental.pallas import tpu as pltpu

D_MODEL = 768
NUM_CLASSES = 10
K = 16
TILE_T = 4096


def _body(h_ref, wc_ref, bc_ref, ws_ref, logits_ref, scores_ref, pooled_ref,
          sc_scores, sc_logits):
    b = pl.program_id(0)
    t = pl.program_id(1)
    B = pl.num_programs(0)
    nt = pl.num_programs(1)
    T = nt * TILE_T

    h = h_ref[0]
    res = jnp.dot(h, wc_ref[...], preferred_element_type=jnp.float32)
    res = res + bc_ref[...]
    logits_ref[0] = res[:, :NUM_CLASSES]
    srow = jax.lax.dot_general(
        ws_ref[...], h, (((1,), (1,)), ((), ())),
        preferred_element_type=jnp.float32) + bc_ref[0, NUM_CLASSES]
    scores_ref[0] = srow
    sc_scores[pl.ds(b, 1), pl.ds(t * TILE_T, TILE_T)] = srow
    sc_logits[pl.ds(b * T + t * TILE_T, TILE_T), :] = res

    @pl.when((b == B - 1) & (t == nt - 1))
    def _pool():
        iota = lax.broadcasted_iota(jnp.int32, (B, T), 1)
        neg = jnp.float32(-jnp.inf)

        def step(_, acc):
            s = sc_scores[...]
            mx = jnp.max(s, axis=1, keepdims=True)
            cand = jnp.where(s == mx, iota, T)
            i = jnp.min(cand, axis=1, keepdims=True)
            sc_scores[...] = jnp.where(iota == i, neg, s)
            rows = []
            for bb in range(B):
                ib = i[bb, 0]
                rows.append(sc_logits[pl.ds(bb * T + ib, 1), :])
            return acc + jnp.concatenate(rows, 0)

        acc = lax.fori_loop(0, K, step, jnp.zeros((B, 16), jnp.float32))
        pooled_ref[0] = acc * (1.0 / K)


def _fused(H, W_cls, b_cls, W_score, b_score):
    B, T, D = H.shape
    nt = T // TILE_T
    wc = jnp.zeros((D, 16), jnp.float32)
    wc = wc.at[:, :NUM_CLASSES].set(W_cls.T)
    wc = wc.at[:, NUM_CLASSES:NUM_CLASSES + 1].set(W_score.T)
    bc = jnp.zeros((1, 16), jnp.float32)
    bc = bc.at[0, :NUM_CLASSES].set(b_cls).at[0, NUM_CLASSES].set(b_score[0])
    return pl.pallas_call(
        _body,
        grid=(B, nt),
        in_specs=[
            pl.BlockSpec((1, TILE_T, D), lambda b, t: (b, t, 0)),
            pl.BlockSpec((D, 16), lambda b, t: (0, 0)),
            pl.BlockSpec((1, 16), lambda b, t: (0, 0)),
            pl.BlockSpec((1, D), lambda b, t: (0, 0)),
        ],
        out_specs=[
            pl.BlockSpec((1, TILE_T, NUM_CLASSES), lambda b, t: (b, t, 0)),
            pl.BlockSpec((1, 1, TILE_T), lambda b, t: (b, 0, t)),
            pl.BlockSpec((1, B, 16), lambda b, t: (0, 0, 0)),
        ],
        out_shape=[
            jax.ShapeDtypeStruct((B, T, NUM_CLASSES), jnp.float32),
            jax.ShapeDtypeStruct((B, 1, T), jnp.float32),
            jax.ShapeDtypeStruct((1, B, 16), jnp.float32),
        ],
        scratch_shapes=[
            pltpu.VMEM((B, T), jnp.float32),
            pltpu.VMEM((B * T, 16), jnp.float32),
        ],
    )(H, wc, bc, W_score)


def kernel(H, W_cls, b_cls, W_score, b_score):
    B, T, _ = H.shape
    logits_t, scores3, pooled16 = _fused(H, W_cls, b_cls, W_score, b_score)
    return (pooled16[0, :, :NUM_CLASSES], logits_t, scores3.reshape(B, T))
